# Initial kernel scaffold; baseline (speedup 1.0000x reference)
import jax
import jax.numpy as jnp
from jax.experimental import pallas as pl


def kernel(batch, x, edge_index, edge_attr, atom_emb, bond_emb, W1, b1, g_mlp, be_mlp, W2, b2, g_out, be_out):
    raise NotImplementedError("write your pallas kernel here")


# speedup vs baseline: 3.8775x; 3.8775x over previous
import functools

import jax
import jax.numpy as jnp
from jax import lax
from jax.experimental import pallas as pl
from jax.experimental.pallas import tpu as pltpu
from jax.experimental.pallas import tpu_sc as plsc

N = 10000
E = 320000
D = 128
D2 = 256
L = 5
NUM_ATOM = 28
NUM_BOND = 4
G = 512

NP = 10240
BLK = 1024
NB = NP // BLK
NC = 2
NS = 16
C = 128
NCH = 80
NHALF = 2
NCHH = NCH // NHALF
EP = NC * NS * NCH * C
NPA = 10112
RPT = NPA // NS
EPS = 1e-5

_INTERPRET = False


def _sc_edge_body(t_hbm, gidx_hbm, dst_hbm, zeros_hbm, out_hbm,
                  gidx_v, dst_v, buf0, buf1, sem0, sem1, accum):
    c = lax.axis_index("c")
    s = lax.axis_index("s")
    pltpu.sync_copy(zeros_hbm, accum.at[pl.ds(s * RPT, RPT)])
    plsc.subcore_barrier()

    for half in range(NHALF):
        pltpu.sync_copy(gidx_hbm.at[c, s, pl.ds(half * NCHH, NCHH)], gidx_v)
        pltpu.sync_copy(dst_hbm.at[c, s, pl.ds(half * NCHH, NCHH)], dst_v)
        pltpu.async_copy(t_hbm.at[gidx_v.at[0]], buf0, sem0)

        def body(jj, carry):
            j0 = 2 * jj
            pltpu.async_copy(t_hbm.at[gidx_v.at[j0 + 1]], buf1, sem1)
            pltpu.make_async_copy(t_hbm.at[gidx_v.at[j0]], buf0, sem0).wait()
            pltpu.sync_copy(buf0, accum.at[dst_v.at[j0]], add=True)

            @pl.when(j0 + 2 < NCHH)
            def _():
                pltpu.async_copy(t_hbm.at[gidx_v.at[j0 + 2]], buf0, sem0)

            pltpu.make_async_copy(t_hbm.at[gidx_v.at[j0 + 1]], buf1, sem1).wait()
            pltpu.sync_copy(buf1, accum.at[dst_v.at[j0 + 1]], add=True)
            return carry

        lax.fori_loop(0, NCHH // 2, body, 0)
    plsc.subcore_barrier()
    pltpu.sync_copy(accum.at[pl.ds(s * RPT, RPT)],
                    out_hbm.at[c, pl.ds(s * RPT, RPT)])


def _sc_edge(t2d, gidx, dst, zeros):
    mesh = plsc.VectorSubcoreMesh(core_axis_name="c", subcore_axis_name="s")
    fn = pl.kernel(
        _sc_edge_body,
        out_type=jax.ShapeDtypeStruct((NC, NP, D), jnp.float32),
        mesh=mesh,
        scratch_types=[
            pltpu.VMEM((NCHH, C), jnp.int32),
            pltpu.VMEM((NCHH, C), jnp.int32),
            pltpu.VMEM((C, D), jnp.float32),
            pltpu.VMEM((C, D), jnp.float32),
            pltpu.SemaphoreType.DMA,
            pltpu.SemaphoreType.DMA,
            pltpu.VMEM_SHARED((NPA, D), jnp.float32),
        ],
    )
    return fn(t2d, gidx, dst, zeros)


def _prep_body(x_ref, atom_ref, bond_ref, h_ref, t_ref):
    b = pl.program_id(0)
    rowid = lax.broadcasted_iota(jnp.int32, (BLK, 1), 0) + b * BLK
    mask = rowid < N
    oh = (x_ref[...] == lax.broadcasted_iota(jnp.int32, (BLK, 32), 1)
          ).astype(jnp.float32)
    h0 = jnp.dot(oh, atom_ref[...], preferred_element_type=jnp.float32,
                 precision=lax.Precision.HIGHEST)
    h_ref[...] = h0
    for bb in range(NUM_BOND):
        t_ref[bb] = jnp.where(
            mask, jax.nn.relu(h0 + bond_ref[bb:bb + 1, :]), 0.0)


def _prep(xp, atom_p, bond):
    return pl.pallas_call(
        _prep_body,
        grid=(NB,),
        in_specs=[
            pl.BlockSpec((BLK, 1), lambda b: (b, 0)),
            pl.BlockSpec((32, D), lambda b: (0, 0)),
            pl.BlockSpec((NUM_BOND, D), lambda b: (0, 0)),
        ],
        out_specs=[
            pl.BlockSpec((BLK, D), lambda b: (b, 0)),
            pl.BlockSpec((NUM_BOND, BLK, D), lambda b: (0, b, 0)),
        ],
        out_shape=[
            jax.ShapeDtypeStruct((NP, D), jnp.float32),
            jax.ShapeDtypeStruct((NUM_BOND, NP, D), jnp.float32),
        ],
        compiler_params=pltpu.CompilerParams(
            dimension_semantics=("arbitrary",)),
        interpret=_INTERPRET,
    )(xp, atom_p, bond)


def _layer_body(last, relu_out,
                h_ref, a_ref, w1_ref, b1_ref, g1_ref, be1_ref,
                w2_ref, b2_ref, g2_ref, be2_ref, aux_ref,
                h_out_ref, t_out_ref, z1s, z2s, st, ps):
    p = pl.program_id(0)
    b = pl.program_id(1)
    rowid = lax.broadcasted_iota(jnp.int32, (BLK, 1), 0) + b * BLK
    mask = rowid < N

    @pl.when(p == 0)
    def _phase0():
        z = h_ref[...] + a_ref[0] + a_ref[1]
        z1 = jnp.dot(z, w1_ref[...],
                     preferred_element_type=jnp.float32) + b1_ref[...]
        z1s[pl.ds(b * BLK, BLK), :] = z1
        zm = jnp.where(mask, z1, 0.0)
        s_ = jnp.sum(zm, axis=0, keepdims=True)
        st[0:1, :] = jnp.where(b == 0, s_, st[0:1, :] + s_)

    @pl.when(p == 1)
    def _phase1():
        @pl.when(b == 0)
        def _():
            st[2:3, :] = st[0:1, :] / N

        mu = st[2:3, :]
        z1 = z1s[pl.ds(b * BLK, BLK), :]
        dev = jnp.where(mask, z1 - mu, 0.0)
        q_ = jnp.sum(dev * dev, axis=0, keepdims=True)
        st[1:2, :] = jnp.where(b == 0, q_, st[1:2, :] + q_)

    @pl.when(p == 2)
    def _phase2():
        @pl.when(b == 0)
        def _():
            st[3:4, :] = jnp.sqrt(st[1:2, :] / N + EPS)

        mu = st[2:3, :]
        s1 = st[3:4, :]
        z1 = z1s[pl.ds(b * BLK, BLK), :]
        z1n = jax.nn.relu((z1 - mu) / s1 * g1_ref[...] + be1_ref[...])
        z2 = jnp.dot(z1n, w2_ref[...],
                     preferred_element_type=jnp.float32) + b2_ref[...]
        z2s[pl.ds(b * BLK, BLK), :] = z2
        zm = jnp.where(mask, z2, 0.0)
        s_ = jnp.sum(zm, axis=0, keepdims=True)
        st[4:5, 0:D] = jnp.where(b == 0, s_, st[4:5, 0:D] + s_)

    @pl.when(p == 3)
    def _phase3():
        @pl.when(b == 0)
        def _():
            st[6:7, 0:D] = st[4:5, 0:D] / N

        mu2 = st[6:7, 0:D]
        z2 = z2s[pl.ds(b * BLK, BLK), :]
        dev = jnp.where(mask, z2 - mu2, 0.0)
        q_ = jnp.sum(dev * dev, axis=0, keepdims=True)
        st[5:6, 0:D] = jnp.where(b == 0, q_, st[5:6, 0:D] + q_)

    @pl.when(p == 4)
    def _phase4():
        @pl.when(b == 0)
        def _():
            st[7:8, 0:D] = jnp.sqrt(st[5:6, 0:D] / N + EPS)

        mu2 = st[6:7, 0:D]
        s2 = st[7:8, 0:D]
        z2 = z2s[pl.ds(b * BLK, BLK), :]
        hn = (z2 - mu2) / s2 * g2_ref[...] + be2_ref[...]
        if relu_out:
            hn = jax.nn.relu(hn)
        hn = jnp.where(mask, hn, 0.0)
        h_out_ref[...] = hn
        if not last:
            for bb in range(NUM_BOND):
                t_out_ref[bb] = jnp.where(
                    mask, jax.nn.relu(hn + aux_ref[bb:bb + 1, :]), 0.0)
        else:
            oh = (aux_ref[...] == lax.broadcasted_iota(jnp.int32, (BLK, G), 1)
                  ).astype(jnp.float32)
            contrib = lax.dot_general(
                oh, hn, (((0,), (0,)), ((), ())),
                preferred_element_type=jnp.float32,
                precision=lax.Precision.HIGHEST)
            ps[...] = jnp.where(b == 0, contrib, ps[...] + contrib)
            t_out_ref[...] = ps[...]


def _layer(h, a, w1, b1, g1, be1, w2, b2, g2, be2, aux, last, relu_out):
    def only0(p, b):
        return (jnp.where(p == 0, b, 0), 0)

    def only0_3d(p, b):
        return (0, jnp.where(p == 0, b, 0), 0)

    def only2(p, b):
        return (jnp.where(p == 4, b, 0), 0)

    def only2_3d(p, b):
        return (0, jnp.where(p == 4, b, 0), 0)

    const2 = lambda p, b: (0, 0)

    if last:
        aux_spec = pl.BlockSpec((BLK, 1), only2)
        t_out_spec = pl.BlockSpec((G, D), const2)
        t_out_shape = jax.ShapeDtypeStruct((G, D), jnp.float32)
    else:
        aux_spec = pl.BlockSpec((NUM_BOND, D), const2)
        t_out_spec = pl.BlockSpec((NUM_BOND, BLK, D), only2_3d)
        t_out_shape = jax.ShapeDtypeStruct((NUM_BOND, NP, D), jnp.float32)

    return pl.pallas_call(
        functools.partial(_layer_body, last, relu_out),
        grid=(5, NB),
        in_specs=[
            pl.BlockSpec((BLK, D), only0),
            pl.BlockSpec((NC, BLK, D), only0_3d),
            pl.BlockSpec((D, D2), const2),
            pl.BlockSpec((1, D2), const2),
            pl.BlockSpec((1, D2), const2),
            pl.BlockSpec((1, D2), const2),
            pl.BlockSpec((D2, D), const2),
            pl.BlockSpec((1, D), const2),
            pl.BlockSpec((1, D), const2),
            pl.BlockSpec((1, D), const2),
            aux_spec,
        ],
        out_specs=[
            pl.BlockSpec((BLK, D), only2),
            t_out_spec,
        ],
        out_shape=[
            jax.ShapeDtypeStruct((NP, D), jnp.float32),
            t_out_shape,
        ],
        scratch_shapes=[
            pltpu.VMEM((NP, D2), jnp.float32),
            pltpu.VMEM((NP, D), jnp.float32),
            pltpu.VMEM((8, D2), jnp.float32),
            pltpu.VMEM((G, D), jnp.float32),
        ],
        compiler_params=pltpu.CompilerParams(
            dimension_semantics=("arbitrary", "arbitrary")),
        interpret=_INTERPRET,
    )(h, a, w1, b1, g1, be1, w2, b2, g2, be2, aux)


def kernel(batch, x, edge_index, edge_attr, atom_emb, bond_emb,
           W1, b1, g_mlp, be_mlp, W2, b2, g_out, be_out):
    f32 = jnp.float32
    i32 = jnp.int32

    xp = jnp.pad(x.astype(i32), ((0, NP - N), (0, 0)), constant_values=28)
    atom_p = jnp.pad(atom_emb, ((0, 32 - NUM_ATOM), (0, 0)))
    batch_p = jnp.pad(batch.astype(i32), (0, NP - N)).reshape(NP, 1)

    src = edge_index[0].astype(i32)
    dst = edge_index[1].astype(i32)
    gidx = edge_attr.astype(i32) * NP + src
    gidx = jnp.pad(gidx, (0, EP - E), constant_values=N)
    dstp = jnp.pad(dst, (0, EP - E)).reshape(NC, NS, NCH, C)
    gidx = gidx.reshape(NC, NS, NCH, C)
    zeros = jnp.zeros((RPT, D), f32)

    b1r = b1.reshape(L, 1, D2)
    g1r = g_mlp.reshape(L, 1, D2)
    be1r = be_mlp.reshape(L, 1, D2)
    b2r = b2.reshape(L, 1, D)
    g2r = g_out.reshape(L, 1, D)
    be2r = be_out.reshape(L, 1, D)

    h, t = _prep(xp, atom_p, bond_emb)
    for i in range(L):
        a = _sc_edge(t.reshape(NUM_BOND * NP, D), gidx, dstp, zeros)
        last = i == L - 1
        aux = batch_p if last else bond_emb
        h, t = _layer(h, a, W1[i], b1r[i], g1r[i], be1r[i],
                      W2[i], b2r[i], g2r[i], be2r[i], aux,
                      last, relu_out=not last)
    return (t, h[:N])
